# baseline (speedup 1.0000x reference)
import jax
import jax.numpy as jnp
from jax.experimental import pallas as pl

N = 8192
STATE_DIM = 512
HID = 128
D_EMB = 64
KSEL = 33
EPS = 0.001
C = 0.001

_ROWS_A = 1024
_ROWS_B = 1024

_INTERPRET = False


def _embed_body(s_ref, w1t_ref, b1_ref, w2t_ref, b2_ref, x_ref, xt_ref):
    h = jnp.dot(s_ref[...], w1t_ref[...], preferred_element_type=jnp.float32)
    h = jnp.maximum(h + b1_ref[...], 0.0)
    x = jnp.dot(h, w2t_ref[...], preferred_element_type=jnp.float32)
    x = x + b2_ref[...]
    x_ref[...] = x
    xt_ref[...] = x.T


def _knn_body(x_ref, xt_ref, out_ref):
    x_r = x_ref[...]
    xt = xt_ref[...]
    sq_r = jnp.sum(x_r * x_r, axis=1, keepdims=True)
    sq_c = jnp.sum(xt * xt, axis=0, keepdims=True)
    d2 = sq_r + sq_c - 2.0 * jnp.dot(x_r, xt, preferred_element_type=jnp.float32)
    d2 = jnp.maximum(d2, 1e-12)
    rows = d2.shape[0]
    ones_col = jnp.ones((N, 1), jnp.float32)

    def bucket_top(b):
        return jax.lax.bitcast_convert_type((b << 14) | 0x3FFF, jnp.float32)

    def count_le(thr):
        mask = jnp.where(d2 <= thr, 1.0, 0.0)
        return jnp.dot(mask, ones_col, preferred_element_type=jnp.float32)

    lo = jnp.full((rows, 1), 0, jnp.int32)
    hi = jnp.full((rows, 1), 0x7F800000 >> 14, jnp.int32)

    def bs_body(_, carry):
        lo, hi = carry
        m2 = lo + ((hi - lo) >> 1)
        m1 = lo + ((m2 - lo) >> 1)
        m3 = m2 + ((hi - m2) >> 1)
        t1 = count_le(bucket_top(m1)) >= KSEL - 0.5
        t2 = count_le(bucket_top(m2)) >= KSEL - 0.5
        t3 = count_le(bucket_top(m3)) >= KSEL - 0.5
        new_lo = jnp.where(t1, lo, jnp.where(t2, m1, jnp.where(t3, m2, m3)))
        new_hi = jnp.where(t1, m1, jnp.where(t2, m2, jnp.where(t3, m3, hi)))
        return new_lo, new_hi

    lo, hi = jax.lax.fori_loop(0, 9, bs_body, (lo, hi))
    cond = d2 <= bucket_top(lo)
    dist = jnp.sqrt(d2)
    sum_lo = jnp.dot(jnp.where(cond, dist, 0.0), ones_col,
                     preferred_element_type=jnp.float32)
    cnt_lo = jnp.dot(jnp.where(cond, 1.0, 0.0), ones_col,
                     preferred_element_type=jnp.float32)
    rep = jax.lax.bitcast_convert_type((hi << 14) | 0x2000, jnp.float32)
    rowmin = jnp.min(d2, axis=1, keepdims=True)
    out_ref[...] = (sum_lo + (KSEL - cnt_lo) * jnp.sqrt(rep)
                    - jnp.sqrt(rowmin))


def _epilogue_body(sk_ref, r_ref):
    sk = sk_ref[...]
    m2 = jnp.mean(sk) ** 2
    knn = EPS / (sk * sk / m2 + EPS)
    r_ref[...] = 1.0 / (jnp.sqrt(knn) + C)


def kernel(s, W1, b1, W2, b2):
    w1t = W1.T
    w2t = W2.T
    b1r = b1.reshape(1, HID)
    b2r = b2.reshape(1, D_EMB)

    x, xt = pl.pallas_call(
        _embed_body,
        grid=(N // _ROWS_A,),
        in_specs=[
            pl.BlockSpec((_ROWS_A, STATE_DIM), lambda i: (i, 0)),
            pl.BlockSpec((STATE_DIM, HID), lambda i: (0, 0)),
            pl.BlockSpec((1, HID), lambda i: (0, 0)),
            pl.BlockSpec((HID, D_EMB), lambda i: (0, 0)),
            pl.BlockSpec((1, D_EMB), lambda i: (0, 0)),
        ],
        out_specs=[
            pl.BlockSpec((_ROWS_A, D_EMB), lambda i: (i, 0)),
            pl.BlockSpec((D_EMB, _ROWS_A), lambda i: (0, i)),
        ],
        out_shape=[
            jax.ShapeDtypeStruct((N, D_EMB), jnp.float32),
            jax.ShapeDtypeStruct((D_EMB, N), jnp.float32),
        ],
        interpret=_INTERPRET,
    )(s, w1t, b1r, w2t, b2r)

    sum_k = pl.pallas_call(
        _knn_body,
        grid=(N // _ROWS_B,),
        in_specs=[
            pl.BlockSpec((_ROWS_B, D_EMB), lambda i: (i, 0)),
            pl.BlockSpec((D_EMB, N), lambda i: (0, 0)),
        ],
        out_specs=pl.BlockSpec((_ROWS_B, 1), lambda i: (i, 0)),
        out_shape=jax.ShapeDtypeStruct((N, 1), jnp.float32),
        interpret=_INTERPRET,
    )(x, xt)

    r = pl.pallas_call(
        _epilogue_body,
        out_shape=jax.ShapeDtypeStruct((N // 128, 128), jnp.float32),
        interpret=_INTERPRET,
    )(sum_k.reshape(N // 128, 128))

    return r.reshape(N, 1)


# speedup vs baseline: 1.4135x; 1.4135x over previous
import jax
import jax.numpy as jnp
from jax.experimental import pallas as pl

N = 8192
STATE_DIM = 512
HID = 128
D_EMB = 64
KSEL = 33
EPS = 0.001
C = 0.001

_ROWS_A = 1024
_ROWS_B = 1024

_INTERPRET = False


def _embed_body(s_ref, w1t_ref, b1_ref, w2t_ref, b2_ref, x_ref, xt_ref):
    h = jnp.dot(s_ref[...], w1t_ref[...], preferred_element_type=jnp.float32)
    h = jnp.maximum(h + b1_ref[...], 0.0)
    x = jnp.dot(h, w2t_ref[...], preferred_element_type=jnp.float32)
    x = x + b2_ref[...]
    x_ref[...] = x
    xt_ref[...] = x.T


def _knn_body(x_ref, xt_ref, out_ref):
    x_r = x_ref[...]
    xt = xt_ref[...]
    sq_r = jnp.sum(x_r * x_r, axis=1, keepdims=True)
    sq_c = jnp.sum(xt * xt, axis=0, keepdims=True)
    d2 = sq_r + sq_c - 2.0 * jnp.dot(x_r, xt, preferred_element_type=jnp.float32)
    d2 = jnp.maximum(d2, 1e-12)
    rows = d2.shape[0]
    rowmin = jnp.min(d2, axis=1, keepdims=True)
    ones_col = jnp.ones((N, 1), jnp.float32)

    def bucket_top(b):
        return jax.lax.bitcast_convert_type((b << 14) | 0x3FFF, jnp.float32)

    def count_le(thr):
        mask = jnp.where(d2 <= thr, 1.0, 0.0)
        return jnp.dot(mask, ones_col, preferred_element_type=jnp.float32)

    lo = jnp.full((rows, 1), 0, jnp.int32)
    hi = jnp.full((rows, 1), 0x7F800000 >> 14, jnp.int32)

    def bs_body(_, carry):
        lo, hi = carry
        mid = lo + ((hi - lo) >> 1)
        take_hi = count_le(bucket_top(mid)) >= KSEL - 0.5
        return jnp.where(take_hi, lo, mid), jnp.where(take_hi, mid, hi)

    lo, hi = jax.lax.fori_loop(0, 17, bs_body, (lo, hi))
    cond = d2 <= bucket_top(lo)
    dist = jnp.sqrt(d2)
    sum_lo = jnp.dot(jnp.where(cond, dist, 0.0), ones_col,
                     preferred_element_type=jnp.float32)
    cnt_lo = jnp.dot(jnp.where(cond, 1.0, 0.0), ones_col,
                     preferred_element_type=jnp.float32)
    rep = jax.lax.bitcast_convert_type((hi << 14) | 0x2000, jnp.float32)
    out_ref[...] = (sum_lo + (KSEL - cnt_lo) * jnp.sqrt(rep)
                    - jnp.sqrt(rowmin))


def _epilogue_body(sk_ref, r_ref):
    sk = sk_ref[...]
    m2 = jnp.mean(sk) ** 2
    knn = EPS / (sk * sk / m2 + EPS)
    r_ref[...] = 1.0 / (jnp.sqrt(knn) + C)


def kernel(s, W1, b1, W2, b2):
    w1t = W1.T
    w2t = W2.T
    b1r = b1.reshape(1, HID)
    b2r = b2.reshape(1, D_EMB)

    x, xt = pl.pallas_call(
        _embed_body,
        grid=(N // _ROWS_A,),
        in_specs=[
            pl.BlockSpec((_ROWS_A, STATE_DIM), lambda i: (i, 0)),
            pl.BlockSpec((STATE_DIM, HID), lambda i: (0, 0)),
            pl.BlockSpec((1, HID), lambda i: (0, 0)),
            pl.BlockSpec((HID, D_EMB), lambda i: (0, 0)),
            pl.BlockSpec((1, D_EMB), lambda i: (0, 0)),
        ],
        out_specs=[
            pl.BlockSpec((_ROWS_A, D_EMB), lambda i: (i, 0)),
            pl.BlockSpec((D_EMB, _ROWS_A), lambda i: (0, i)),
        ],
        out_shape=[
            jax.ShapeDtypeStruct((N, D_EMB), jnp.float32),
            jax.ShapeDtypeStruct((D_EMB, N), jnp.float32),
        ],
        interpret=_INTERPRET,
    )(s, w1t, b1r, w2t, b2r)

    sum_k = pl.pallas_call(
        _knn_body,
        grid=(N // _ROWS_B,),
        in_specs=[
            pl.BlockSpec((_ROWS_B, D_EMB), lambda i: (i, 0)),
            pl.BlockSpec((D_EMB, N), lambda i: (0, 0)),
        ],
        out_specs=pl.BlockSpec((_ROWS_B, 1), lambda i: (i, 0)),
        out_shape=jax.ShapeDtypeStruct((N, 1), jnp.float32),
        interpret=_INTERPRET,
    )(x, xt)

    r = pl.pallas_call(
        _epilogue_body,
        out_shape=jax.ShapeDtypeStruct((N // 128, 128), jnp.float32),
        interpret=_INTERPRET,
    )(sum_k.reshape(N // 128, 128))

    return r.reshape(N, 1)


# speedup vs baseline: 1.5634x; 1.1060x over previous
import jax
import jax.numpy as jnp
from jax.experimental import pallas as pl

N = 8192
STATE_DIM = 512
HID = 128
D_EMB = 64
KSEL = 33
EPS = 0.001
C = 0.001

_ROWS_A = 1024
_ROWS_B = 1024

_INTERPRET = False


def _embed_body(s_ref, w1t_ref, b1_ref, w2t_ref, b2_ref, x_ref, xt_ref):
    h = jnp.dot(s_ref[...], w1t_ref[...], preferred_element_type=jnp.float32)
    h = jnp.maximum(h + b1_ref[...], 0.0)
    x = jnp.dot(h, w2t_ref[...], preferred_element_type=jnp.float32)
    x = x + b2_ref[...]
    x_ref[...] = x
    xt_ref[...] = x.T


def _knn_body(x_ref, xt_ref, out_ref):
    x_r = x_ref[...]
    xt = xt_ref[...]
    sq_r = jnp.sum(x_r * x_r, axis=1, keepdims=True)
    sq_c = jnp.sum(xt * xt, axis=0, keepdims=True)
    d2 = sq_r + sq_c - 2.0 * jnp.dot(x_r, xt, preferred_element_type=jnp.float32)
    d2 = jnp.maximum(d2, 1e-12)
    rows = d2.shape[0]
    rowmin = jnp.min(d2, axis=1, keepdims=True)
    ones_col = jnp.ones((N, 1), jnp.float32)

    def bucket_top(b):
        return jax.lax.bitcast_convert_type((b << 16) | 0xFFFF, jnp.float32)

    def count_le(thr):
        mask = jnp.where(d2 <= thr, 1.0, 0.0)
        return jnp.dot(mask, ones_col, preferred_element_type=jnp.float32)

    lo = jnp.full((rows, 1), 0, jnp.int32)
    hi = jnp.full((rows, 1), 0x7F800000 >> 16, jnp.int32)

    def bs_body(_, carry):
        lo, hi = carry
        mid = lo + ((hi - lo) >> 1)
        take_hi = count_le(bucket_top(mid)) >= KSEL - 0.5
        return jnp.where(take_hi, lo, mid), jnp.where(take_hi, mid, hi)

    lo, hi = jax.lax.fori_loop(0, 15, bs_body, (lo, hi))
    cond = d2 <= bucket_top(lo)
    dist = jnp.sqrt(d2)
    sum_lo = jnp.dot(jnp.where(cond, dist, 0.0), ones_col,
                     preferred_element_type=jnp.float32)
    cnt_lo = jnp.dot(jnp.where(cond, 1.0, 0.0), ones_col,
                     preferred_element_type=jnp.float32)
    rep = jax.lax.bitcast_convert_type((hi << 16) | 0x8000, jnp.float32)
    out_ref[...] = (sum_lo + (KSEL - cnt_lo) * jnp.sqrt(rep)
                    - jnp.sqrt(rowmin))


def _epilogue_body(sk_ref, r_ref):
    sk = sk_ref[...]
    m2 = jnp.mean(sk) ** 2
    knn = EPS / (sk * sk / m2 + EPS)
    r_ref[...] = 1.0 / (jnp.sqrt(knn) + C)


def kernel(s, W1, b1, W2, b2):
    w1t = W1.T
    w2t = W2.T
    b1r = b1.reshape(1, HID)
    b2r = b2.reshape(1, D_EMB)

    x, xt = pl.pallas_call(
        _embed_body,
        grid=(N // _ROWS_A,),
        in_specs=[
            pl.BlockSpec((_ROWS_A, STATE_DIM), lambda i: (i, 0)),
            pl.BlockSpec((STATE_DIM, HID), lambda i: (0, 0)),
            pl.BlockSpec((1, HID), lambda i: (0, 0)),
            pl.BlockSpec((HID, D_EMB), lambda i: (0, 0)),
            pl.BlockSpec((1, D_EMB), lambda i: (0, 0)),
        ],
        out_specs=[
            pl.BlockSpec((_ROWS_A, D_EMB), lambda i: (i, 0)),
            pl.BlockSpec((D_EMB, _ROWS_A), lambda i: (0, i)),
        ],
        out_shape=[
            jax.ShapeDtypeStruct((N, D_EMB), jnp.float32),
            jax.ShapeDtypeStruct((D_EMB, N), jnp.float32),
        ],
        interpret=_INTERPRET,
    )(s, w1t, b1r, w2t, b2r)

    sum_k = pl.pallas_call(
        _knn_body,
        grid=(N // _ROWS_B,),
        in_specs=[
            pl.BlockSpec((_ROWS_B, D_EMB), lambda i: (i, 0)),
            pl.BlockSpec((D_EMB, N), lambda i: (0, 0)),
        ],
        out_specs=pl.BlockSpec((_ROWS_B, 1), lambda i: (i, 0)),
        out_shape=jax.ShapeDtypeStruct((N, 1), jnp.float32),
        interpret=_INTERPRET,
    )(x, xt)

    r = pl.pallas_call(
        _epilogue_body,
        out_shape=jax.ShapeDtypeStruct((N // 128, 128), jnp.float32),
        interpret=_INTERPRET,
    )(sum_k.reshape(N // 128, 128))

    return r.reshape(N, 1)
